# baseline (speedup 1.0000x reference)
import functools

import jax
import jax.numpy as jnp
from jax.experimental import pallas as pl
import jax.experimental.pallas.tpu as pltpu

N_HITS = 50000
D = 512
C = 32
K = 128
Q_MIN = 0.5
B = 10000
NB = N_HITS // B
NEG = -1e30
BIGIDX = 1e9
IBIG = 1 << 30

_INTERPRET = False


def _sigmoid(v):
    return 1.0 / (1.0 + jnp.exp(-v))


def _q_of(of):
    beta_c = jnp.clip(of, 0.0, 1.0 - 1e-4)
    at = 0.5 * jnp.log((1.0 + beta_c) / (1.0 - beta_c))
    return at * at + Q_MIN


def _body(x_ref, w_ref, b_ref, y_ref, out_ref,
          ox_s, x2_s, q_s, xal_s, bal_s, sidx_s, of0_s, ox0_s, acc_s):
    p = pl.program_id(0)
    j = pl.program_id(1)

    @pl.when(p == 0)
    def _phase0():
        xb = x_ref[...]
        h = jax.lax.dot_general(
            w_ref[...], xb, (((0,), (1,)), ((), ())),
            preferred_element_type=jnp.float32) + b_ref[...]
        ox = h[:C, :]
        ofl = _sigmoid(h[C:C + 1, :])
        ox_hi_bf = ox.astype(jnp.bfloat16)
        ox_s[j] = ox_hi_bf
        x2_s[j] = jnp.sum(ox * ox, axis=0, keepdims=True)
        q_s[j] = _q_of(ofl)

        y = y_ref[0]
        inst = jax.lax.broadcasted_iota(jnp.int32, (K, 1), 0)
        mask = (y == inst)
        ofb = jnp.where(mask, ofl, NEG)
        bmax = jnp.max(ofb, axis=1, keepdims=True)
        lanes = jax.lax.broadcasted_iota(jnp.int32, (1, B), 1)
        idxm = jnp.where(ofb == bmax, lanes, lanes + IBIG)
        bidx_l = jnp.min(idxm, axis=1, keepdims=True)
        sel = (idxm == bidx_l).astype(jnp.float32)
        bidx = (bidx_l + j * B).astype(jnp.float32)
        ox_hi = ox_hi_bf.astype(jnp.float32)
        ox_lo = ox - ox_hi
        xal_blk = (
            jax.lax.dot_general(sel, ox_hi, (((1,), (1,)), ((), ())),
                                preferred_element_type=jnp.float32)
            + jax.lax.dot_general(sel, ox_lo, (((1,), (1,)), ((), ())),
                                  preferred_element_type=jnp.float32))

        @pl.when(j == 0)
        def _init():
            bal_s[...] = jnp.full((K, 1), NEG, jnp.float32)
            sidx_s[...] = jnp.full((K, 1), BIGIDX, jnp.float32)
            xal_s[...] = jnp.zeros((K, C), jnp.float32)
            of0_s[...] = jnp.broadcast_to(ofl[0:1, 0:1], (K, 1))
            first = (lanes == 0).astype(jnp.float32)
            ox0_s[...] = (
                jax.lax.dot_general(first, ox_hi, (((1,), (1,)), ((), ())),
                                    preferred_element_type=jnp.float32)
                + jax.lax.dot_general(first, ox_lo, (((1,), (1,)), ((), ())),
                                      preferred_element_type=jnp.float32))

        smax = bal_s[...]
        sidx = sidx_s[...]
        better = bmax > smax
        equal = bmax == smax
        take = better | (equal & (bidx < sidx))
        xal_s[...] = jnp.where(take, xal_blk, xal_s[...])
        sidx_s[...] = jnp.where(
            better, bidx, jnp.where(equal, jnp.minimum(sidx, bidx), sidx))
        bal_s[...] = jnp.maximum(smax, bmax)

        @pl.when(j == NB - 1)
        def _finalize():
            smax2 = bal_s[...]
            empty = smax2 <= NEG / 2
            bal_s[...] = jnp.where(empty, of0_s[...], smax2)
            xal_s[...] = jnp.where(empty,
                                   jnp.broadcast_to(ox0_s[...], (K, C)),
                                   xal_s[...])

    @pl.when(p == 1)
    def _phase1():
        @pl.when(j == 0)
        def _init():
            acc_s[...] = jnp.zeros((K, 1), jnp.float32)

        ox_bf = ox_s[j]
        q = q_s[j]
        y = y_ref[0]
        xal = xal_s[...]
        bal = bal_s[...]
        qal = _q_of(bal)

        dotp2 = jax.lax.dot_general(
            (2.0 * xal).astype(jnp.bfloat16), ox_bf, (((1,), (0,)), ((), ())),
            preferred_element_type=jnp.float32)
        x2 = x2_s[j]
        a2 = jnp.sum(xal * xal, axis=1, keepdims=True)
        d2 = jnp.maximum((x2 + a2) - dotp2, 0.0)
        inst = jax.lax.broadcasted_iota(jnp.int32, (K, 1), 0)
        mask = (y == inst)
        attr = q * jnp.where(mask, d2, 0.0)
        acc_s[...] += qal * jnp.sum(attr, axis=1, keepdims=True)
        d2min = jnp.min(d2)

        @pl.when(d2min < 1.0)
        def _rep():
            u = jnp.maximum(1.0 - jnp.sqrt(d2), 0.0)
            rep = q * jnp.where(mask, 0.0, u)
            acc_s[...] += qal * jnp.sum(rep, axis=1, keepdims=True)

        @pl.when(j == NB - 1)
        def _done():
            l_v = jnp.sum(acc_s[...]) / N_HITS
            l_beta = 1.0 - jnp.sum(bal_s[...]) / K
            out_ref[...] = jnp.reshape(l_v + l_beta, (1, 1))


@functools.partial(jax.jit, static_argnames=())
def kernel(x, y_instance, beta_w, beta_b, coord_w, coord_b, temp):
    w = jnp.concatenate([coord_w, beta_w], axis=1)
    bias = jnp.concatenate([coord_b, beta_b]).reshape(C + 1, 1)
    y3 = y_instance.astype(jnp.int32).reshape(NB, 1, B)

    lvlb = pl.pallas_call(
        _body,
        grid=(2, NB),
        in_specs=[
            pl.BlockSpec((B, D), lambda p, j: (j * (1 - p) + (NB - 1) * p, 0)),
            pl.BlockSpec((D, C + 1), lambda p, j: (0, 0)),
            pl.BlockSpec((C + 1, 1), lambda p, j: (0, 0)),
            pl.BlockSpec((1, 1, B), lambda p, j: (j, 0, 0)),
        ],
        out_specs=pl.BlockSpec((1, 1), lambda p, j: (0, 0)),
        out_shape=jax.ShapeDtypeStruct((1, 1), jnp.float32),
        scratch_shapes=[
            pltpu.VMEM((NB, C, B), jnp.bfloat16),
            pltpu.VMEM((NB, 1, B), jnp.float32),
            pltpu.VMEM((NB, 1, B), jnp.float32),
            pltpu.VMEM((K, C), jnp.float32),
            pltpu.VMEM((K, 1), jnp.float32),
            pltpu.VMEM((K, 1), jnp.float32),
            pltpu.VMEM((K, 1), jnp.float32),
            pltpu.VMEM((1, C), jnp.float32),
            pltpu.VMEM((K, 1), jnp.float32),
        ],
        interpret=_INTERPRET,
    )(x, w, bias, y3)

    return jnp.exp(-temp) * lvlb[0, 0] + temp


# speedup vs baseline: 1.0991x; 1.0991x over previous
import functools

import jax
import jax.numpy as jnp
from jax.experimental import pallas as pl
import jax.experimental.pallas.tpu as pltpu

N_HITS = 50000
D = 512
C = 32
K = 128
Q_MIN = 0.5
B = 10000
NB = N_HITS // B
NEG = -1e30
BIGIDX = 1e9
IBIG = 1 << 30

_INTERPRET = False


def _sigmoid(v):
    return 1.0 / (1.0 + jnp.exp(-v))


def _q_of(of):
    beta_c = jnp.clip(of, 0.0, 1.0 - 1e-4)
    at = 0.5 * jnp.log((1.0 + beta_c) / (1.0 - beta_c))
    return at * at + Q_MIN


def _body(x_ref, w_ref, b_ref, y_ref, out_ref,
          ox_s, x2_s, q_s, xal_s, bal_s, sidx_s, of0_s, ox0_s, acc_s):
    p = pl.program_id(0)
    j = pl.program_id(1)

    @pl.when(p == 0)
    def _phase0():
        xb = x_ref[...]
        h = jax.lax.dot_general(
            w_ref[...], xb, (((0,), (1,)), ((), ())),
            preferred_element_type=jnp.float32) + b_ref[...]
        ox = h[:C, :]
        ofl = _sigmoid(h[C:C + 1, :])
        ox_hi_bf = ox.astype(jnp.bfloat16)
        ox_s[j] = ox_hi_bf
        x2_s[j] = jnp.sum(ox * ox, axis=0, keepdims=True)
        q_s[j] = _q_of(ofl)

        y = y_ref[0]
        inst = jax.lax.broadcasted_iota(jnp.int32, (K, 1), 0)
        mask = (y == inst)
        ofb = jnp.where(mask, ofl, NEG)
        bmax = jnp.max(ofb, axis=1, keepdims=True)
        lanes = jax.lax.broadcasted_iota(jnp.int32, (1, B), 1)
        idxm = jnp.where(ofb == bmax, lanes, lanes + IBIG)
        bidx_l = jnp.min(idxm, axis=1, keepdims=True)
        sel = (idxm == bidx_l).astype(jnp.float32)
        bidx = (bidx_l + j * B).astype(jnp.float32)
        ox_hi = ox_hi_bf.astype(jnp.float32)
        ox_lo = ox - ox_hi
        xal_blk = (
            jax.lax.dot_general(sel, ox_hi, (((1,), (1,)), ((), ())),
                                preferred_element_type=jnp.float32)
            + jax.lax.dot_general(sel, ox_lo, (((1,), (1,)), ((), ())),
                                  preferred_element_type=jnp.float32))

        @pl.when(j == 0)
        def _init():
            bal_s[...] = jnp.full((K, 1), NEG, jnp.float32)
            sidx_s[...] = jnp.full((K, 1), BIGIDX, jnp.float32)
            xal_s[...] = jnp.zeros((K, C), jnp.float32)
            of0_s[...] = jnp.broadcast_to(ofl[0:1, 0:1], (K, 1))
            first = (lanes == 0).astype(jnp.float32)
            ox0_s[...] = (
                jax.lax.dot_general(first, ox_hi, (((1,), (1,)), ((), ())),
                                    preferred_element_type=jnp.float32)
                + jax.lax.dot_general(first, ox_lo, (((1,), (1,)), ((), ())),
                                      preferred_element_type=jnp.float32))

        smax = bal_s[...]
        sidx = sidx_s[...]
        better = bmax > smax
        equal = bmax == smax
        take = better | (equal & (bidx < sidx))
        xal_s[...] = jnp.where(take, xal_blk, xal_s[...])
        sidx_s[...] = jnp.where(
            better, bidx, jnp.where(equal, jnp.minimum(sidx, bidx), sidx))
        bal_s[...] = jnp.maximum(smax, bmax)

        @pl.when(j == NB - 1)
        def _finalize():
            smax2 = bal_s[...]
            empty = smax2 <= NEG / 2
            bal_s[...] = jnp.where(empty, of0_s[...], smax2)
            xal_s[...] = jnp.where(empty,
                                   jnp.broadcast_to(ox0_s[...], (K, C)),
                                   xal_s[...])

    @pl.when(p == 1)
    def _phase1():
        @pl.when(j == 0)
        def _init():
            acc_s[...] = jnp.zeros((K, 1), jnp.float32)

        ox_bf = ox_s[j]
        q = q_s[j]
        y = y_ref[0]
        xal = xal_s[...]
        bal = bal_s[...]
        qal = _q_of(bal)

        dotp2 = jax.lax.dot_general(
            (2.0 * xal).astype(jnp.bfloat16), ox_bf, (((1,), (0,)), ((), ())),
            preferred_element_type=jnp.float32)
        x2 = x2_s[j]
        a2 = jnp.sum(xal * xal, axis=1, keepdims=True)
        d2 = jnp.maximum((x2 + a2) - dotp2, 0.0)
        dist = jnp.sqrt(d2)
        inst = jax.lax.broadcasted_iota(jnp.int32, (K, 1), 0)
        mask = (y == inst)
        contrib = q * jnp.where(mask, d2, jnp.maximum(1.0 - dist, 0.0))
        acc_s[...] += qal * jnp.sum(contrib, axis=1, keepdims=True)

        @pl.when(j == NB - 1)
        def _done():
            l_v = jnp.sum(acc_s[...]) / N_HITS
            l_beta = 1.0 - jnp.sum(bal_s[...]) / K
            out_ref[...] = jnp.reshape(l_v + l_beta, (1, 1))


@functools.partial(jax.jit, static_argnames=())
def kernel(x, y_instance, beta_w, beta_b, coord_w, coord_b, temp):
    w = jnp.concatenate([coord_w, beta_w], axis=1)
    bias = jnp.concatenate([coord_b, beta_b]).reshape(C + 1, 1)
    y3 = y_instance.astype(jnp.int32).reshape(NB, 1, B)

    lvlb = pl.pallas_call(
        _body,
        grid=(2, NB),
        in_specs=[
            pl.BlockSpec((B, D), lambda p, j: (j * (1 - p) + (NB - 1) * p, 0)),
            pl.BlockSpec((D, C + 1), lambda p, j: (0, 0)),
            pl.BlockSpec((C + 1, 1), lambda p, j: (0, 0)),
            pl.BlockSpec((1, 1, B), lambda p, j: (j, 0, 0)),
        ],
        out_specs=pl.BlockSpec((1, 1), lambda p, j: (0, 0)),
        out_shape=jax.ShapeDtypeStruct((1, 1), jnp.float32),
        scratch_shapes=[
            pltpu.VMEM((NB, C, B), jnp.bfloat16),
            pltpu.VMEM((NB, 1, B), jnp.float32),
            pltpu.VMEM((NB, 1, B), jnp.float32),
            pltpu.VMEM((K, C), jnp.float32),
            pltpu.VMEM((K, 1), jnp.float32),
            pltpu.VMEM((K, 1), jnp.float32),
            pltpu.VMEM((K, 1), jnp.float32),
            pltpu.VMEM((1, C), jnp.float32),
            pltpu.VMEM((K, 1), jnp.float32),
        ],
        interpret=_INTERPRET,
    )(x, w, bias, y3)

    return jnp.exp(-temp) * lvlb[0, 0] + temp
